# Initial kernel scaffold; baseline (speedup 1.0000x reference)
import jax
import jax.numpy as jnp
from jax.experimental import pallas as pl


def kernel(inputs, emb_weight):
    raise NotImplementedError("write your pallas kernel here")


# speedup vs baseline: 2.4516x; 2.4516x over previous
import functools

import jax
import jax.numpy as jnp
from jax import lax
from jax.experimental import pallas as pl
from jax.experimental.pallas import tpu as pltpu
from jax.experimental.pallas import tpu_sc as plsc

EMB_DIM = 32


def _gather_sc(idx_flat, emb_weight, n):
    info = plsc.get_sparse_core_info()
    nc, ns = info.num_cores, info.num_subcores
    nw = nc * ns
    per_w = n // nw
    chunk = 1664
    n_chunks = per_w // chunk
    assert per_w % chunk == 0

    mesh = plsc.VectorSubcoreMesh(core_axis_name="c", subcore_axis_name="s")

    @functools.partial(
        pl.kernel,
        mesh=mesh,
        compiler_params=pltpu.CompilerParams(use_tc_tiling_on_sc=False),
        out_type=jax.ShapeDtypeStruct((n, EMB_DIM), jnp.float32),
        scratch_types=[
            pltpu.VMEM((chunk,), jnp.int32),
            pltpu.VMEM((chunk, EMB_DIM), jnp.float32),
            pltpu.SemaphoreType.DMA,
        ],
    )
    def gather_kernel(idx_hbm, table_hbm, out_hbm, idx_v, rows_v, sem):
        wid = lax.axis_index("s") * nc + lax.axis_index("c")
        base = wid * per_w

        def body(c, carry):
            off = base + c * chunk
            pltpu.sync_copy(idx_hbm.at[pl.ds(off, chunk)], idx_v)
            pltpu.async_copy(table_hbm.at[idx_v], rows_v, sem).wait()
            pltpu.sync_copy(rows_v, out_hbm.at[pl.ds(off, chunk)])
            return carry

        lax.fori_loop(0, n_chunks, body, 0)

    return gather_kernel(idx_flat, emb_weight)


def _transpose_tc(rows, b, hw):
    def tkernel(x_ref, o_ref):
        o_ref[...] = jnp.transpose(x_ref[...], (0, 2, 1))

    return pl.pallas_call(
        tkernel,
        grid=(b,),
        in_specs=[pl.BlockSpec((1, hw, EMB_DIM), lambda i: (i, 0, 0))],
        out_specs=pl.BlockSpec((1, EMB_DIM, hw), lambda i: (i, 0, 0)),
        out_shape=jax.ShapeDtypeStruct((b, EMB_DIM, hw), jnp.float32),
    )(rows)


def kernel(inputs, emb_weight):
    b, h, w = inputs.shape
    n = b * h * w
    idx_flat = inputs.reshape(-1).astype(jnp.int32)
    rows = _gather_sc(idx_flat, emb_weight, n)
    rows = rows.reshape(b, h * w, EMB_DIM)
    out = _transpose_tc(rows, b, h * w)
    return out.reshape(b, EMB_DIM, h, w)


# speedup vs baseline: 2.8150x; 1.1482x over previous
import functools

import jax
import jax.numpy as jnp
from jax import lax
from jax.experimental import pallas as pl
from jax.experimental.pallas import tpu as pltpu
from jax.experimental.pallas import tpu_sc as plsc

EMB_DIM = 32


def _embed_sc(idx_flat, emb_weight, b, hw):
    info = plsc.get_sparse_core_info()
    nc, ns = info.num_cores, info.num_subcores
    nw = nc * ns
    assert b % (2 * nw) == 0
    pairs_per_w = b // (2 * nw)
    n_jg_full = hw // 16
    tail = hw - n_jg_full * 16
    hw_pad = (n_jg_full + 1) * 16
    rows = 2 * hw
    rows_pad = 2 * hw_pad

    mesh = plsc.VectorSubcoreMesh(core_axis_name="c", subcore_axis_name="s")

    @functools.partial(
        pl.kernel,
        mesh=mesh,
        compiler_params=pltpu.CompilerParams(
            use_tc_tiling_on_sc=False, needs_layout_passes=False
        ),
        out_type=jax.ShapeDtypeStruct((b, EMB_DIM, hw), jnp.float32),
        scratch_types=[
            pltpu.VMEM((rows,), jnp.int32),
            pltpu.VMEM((rows_pad, EMB_DIM), jnp.float32),
            pltpu.VMEM((EMB_DIM, hw), jnp.float32),
            pltpu.SemaphoreType.DMA,
        ],
    )
    def embed_kernel(idx_hbm, table_hbm, out_hbm, idx_v, rows_v, out_v, sem):
        wid = lax.axis_index("s") * nc + lax.axis_index("c")
        pair0 = wid * pairs_per_w
        iota16 = lax.iota(jnp.int32, 16)

        def pair_body(p, carry):
            pair = pair0 + p
            pltpu.sync_copy(idx_hbm.at[pl.ds(pair * rows, rows)], idx_v)
            pltpu.async_copy(
                table_hbm.at[idx_v], rows_v.at[pl.ds(0, rows)], sem
            ).wait()
            for half in range(2):

                def jg_body(jg, c):
                    row_ids = half * hw + jg * 16 + iota16
                    for d in range(EMB_DIM):
                        col_ids = jnp.full((16,), d, jnp.int32)
                        v = plsc.load_gather(rows_v, [row_ids, col_ids])
                        out_v[d, pl.ds(jg * 16, 16)] = v
                    return c

                lax.fori_loop(0, n_jg_full, jg_body, 0)
                trow_ids = half * hw + n_jg_full * 16 + iota16
                tcol_ids = n_jg_full * 16 + iota16
                tmask = iota16 < tail
                for d in range(EMB_DIM):
                    col_ids = jnp.full((16,), d, jnp.int32)
                    v = plsc.load_gather(rows_v, [trow_ids, col_ids])
                    plsc.store_scatter(
                        out_v, [col_ids, tcol_ids], v, mask=tmask
                    )
                pltpu.sync_copy(out_v, out_hbm.at[pair * 2 + half])
            return carry

        lax.fori_loop(0, pairs_per_w, pair_body, 0)

    return embed_kernel(idx_flat, emb_weight)


def kernel(inputs, emb_weight):
    b, h, w = inputs.shape
    idx_flat = inputs.reshape(-1).astype(jnp.int32)
    out = _embed_sc(idx_flat, emb_weight, b, h * w)
    return out.reshape(b, EMB_DIM, h, w)


# speedup vs baseline: 3.0020x; 1.0664x over previous
import functools

import jax
import jax.numpy as jnp
from jax import lax
from jax.experimental import pallas as pl
from jax.experimental.pallas import tpu as pltpu
from jax.experimental.pallas import tpu_sc as plsc

EMB_DIM = 32
HW = 1300
HWP = 1312
CHUNK = HWP // 2
N_JG = CHUNK // 16


def _embed_sc(idx_pad, emb_weight, b):
    info = plsc.get_sparse_core_info()
    nc, ns = info.num_cores, info.num_subcores
    nw = nc * ns
    assert b % (2 * nw) == 0
    b_per_w = b // nw
    n_chunks = 2 * b_per_w

    mesh = plsc.VectorSubcoreMesh(core_axis_name="c", subcore_axis_name="s")

    @functools.partial(
        pl.kernel,
        mesh=mesh,
        compiler_params=pltpu.CompilerParams(
            use_tc_tiling_on_sc=False, needs_layout_passes=False
        ),
        out_type=jax.ShapeDtypeStruct((b, EMB_DIM, HW), jnp.float32),
        scratch_types=[
            pltpu.VMEM((CHUNK,), jnp.int32),
            pltpu.VMEM((CHUNK,), jnp.int32),
            pltpu.VMEM((CHUNK, EMB_DIM), jnp.float32),
            pltpu.VMEM((CHUNK, EMB_DIM), jnp.float32),
            pltpu.VMEM((EMB_DIM, HW), jnp.float32),
            pltpu.VMEM((EMB_DIM, HW), jnp.float32),
            pltpu.SemaphoreType.DMA,
            pltpu.SemaphoreType.DMA,
            pltpu.SemaphoreType.DMA,
            pltpu.SemaphoreType.DMA,
            pltpu.SemaphoreType.DMA,
            pltpu.SemaphoreType.DMA,
        ],
    )
    def embed_kernel(
        idx_hbm,
        table_hbm,
        out_hbm,
        idx_b0,
        idx_b1,
        rows_b0,
        rows_b1,
        out_b0,
        out_b1,
        isem0,
        isem1,
        gsem0,
        gsem1,
        osem0,
        osem1,
    ):
        idx_b = (idx_b0, idx_b1)
        rows_b = (rows_b0, rows_b1)
        out_b = (out_b0, out_b1)
        isem = (isem0, isem1)
        gsem = (gsem0, gsem1)
        osem = (osem0, osem1)

        wid = lax.axis_index("s") * nc + lax.axis_index("c")
        base_b = wid * b_per_w
        base_i = base_b * HWP
        iota16 = lax.iota(jnp.int32, 16)

        def start_idx(s, p):
            pltpu.async_copy(
                idx_hbm.at[pl.ds(base_i + s * CHUNK, CHUNK)], idx_b[p], isem[p]
            )

        def start_gather(s, p):
            del s
            pltpu.async_copy(table_hbm.at[idx_b[p]], rows_b[p], gsem[p])

        def wait_idx(p):
            pltpu.make_async_copy(
                idx_hbm.at[pl.ds(0, CHUNK)], idx_b[p], isem[p]
            ).wait()

        def wait_gather(p):
            pltpu.make_async_copy(
                table_hbm.at[idx_b[p]], rows_b[p], gsem[p]
            ).wait()

        def start_out(bb, q):
            pltpu.async_copy(out_b[q], out_hbm.at[bb], osem[q])

        def wait_out(bb, q):
            pltpu.make_async_copy(out_b[q], out_hbm.at[bb], osem[q]).wait()

        def transpose_chunk(rb, ob, c):
            col0 = c * CHUNK
            n_full = N_JG if c == 0 else N_JG - 1

            def jg_body(jg, carry):
                row_ids = jg * 16 + iota16
                for d in range(EMB_DIM):
                    col_ids = jnp.full((16,), d, jnp.int32)
                    v = plsc.load_gather(rb, [row_ids, col_ids])
                    ob[d, pl.ds(col0 + jg * 16, 16)] = v
                return carry

            lax.fori_loop(0, n_full, jg_body, 0)
            if c == 1:
                trow_ids = (N_JG - 1) * 16 + iota16
                tcol_ids = col0 + (N_JG - 1) * 16 + iota16
                tmask = iota16 < (HW - col0 - (N_JG - 1) * 16)
                for d in range(EMB_DIM):
                    col_ids = jnp.full((16,), d, jnp.int32)
                    v = plsc.load_gather(rb, [trow_ids, col_ids])
                    plsc.store_scatter(ob, [col_ids, tcol_ids], v, mask=tmask)

        start_idx(0, 0)
        wait_idx(0)
        start_gather(0, 0)
        start_idx(1, 1)

        def loop_body(i, carry):
            for s_local in range(4):
                s = 4 * i + s_local
                p = s_local & 1
                q = (s_local >> 1) & 1
                bb = base_b + 2 * i + (s_local >> 1)

                p1 = (s_local + 1) & 1

                @pl.when(s + 1 < n_chunks)
                def _():
                    wait_idx(p1)
                    start_gather(s + 1, p1)

                @pl.when(s + 2 < n_chunks)
                def _():
                    start_idx(s + 2, p)

                wait_gather(p)
                if s_local in (0, 2):
                    @pl.when(i >= 1)
                    def _():
                        wait_out(bb, q)

                transpose_chunk(rows_b[p], out_b[q], p)
                if s_local in (1, 3):
                    start_out(bb, q)
            return carry

        lax.fori_loop(0, b_per_w // 2, loop_body, 0)

        wait_out(base_b + b_per_w - 2, 0)
        wait_out(base_b + b_per_w - 1, 1)

    return embed_kernel(idx_pad, emb_weight)


def kernel(inputs, emb_weight):
    b, h, w = inputs.shape
    assert h * w == HW
    idx = inputs.reshape(b, HW).astype(jnp.int32)
    idx_pad = jnp.pad(idx, ((0, 0), (0, HWP - HW))).reshape(-1)
    out = _embed_sc(idx_pad, emb_weight, b)
    return out.reshape(b, EMB_DIM, h, w)


# speedup vs baseline: 4.6236x; 1.5402x over previous
import functools

import jax
import jax.numpy as jnp
from jax import lax
from jax.experimental import pallas as pl
from jax.experimental.pallas import tpu as pltpu
from jax.experimental.pallas import tpu_sc as plsc

EMB_DIM = 32
HW = 1300
HWP = 1312
CHUNK = HWP // 2
N_JG = CHUNK // 16


def _embed_sc(idx_pad, emb_weight, b):
    info = plsc.get_sparse_core_info()
    nc, ns = info.num_cores, info.num_subcores
    nw = nc * ns
    assert b % (2 * nw) == 0
    b_per_w = b // nw
    n_chunks = 2 * b_per_w

    mesh = plsc.VectorSubcoreMesh(core_axis_name="c", subcore_axis_name="s")

    @functools.partial(
        pl.kernel,
        mesh=mesh,
        compiler_params=pltpu.CompilerParams(
            use_tc_tiling_on_sc=False, needs_layout_passes=False
        ),
        out_type=jax.ShapeDtypeStruct((b, EMB_DIM, HW), jnp.float32),
        scratch_types=[
            pltpu.VMEM((CHUNK,), jnp.int32),
            pltpu.VMEM((CHUNK,), jnp.int32),
            pltpu.VMEM((CHUNK, EMB_DIM), jnp.float32),
            pltpu.VMEM((CHUNK, EMB_DIM), jnp.float32),
            pltpu.VMEM((EMB_DIM, HW), jnp.float32),
            pltpu.VMEM((EMB_DIM, HW), jnp.float32),
            pltpu.SemaphoreType.DMA,
            pltpu.SemaphoreType.DMA,
            pltpu.SemaphoreType.DMA,
            pltpu.SemaphoreType.DMA,
            pltpu.SemaphoreType.DMA,
            pltpu.SemaphoreType.DMA,
        ],
    )
    def embed_kernel(
        idx_hbm,
        table_hbm,
        out_hbm,
        idx_b0,
        idx_b1,
        rows_b0,
        rows_b1,
        out_b0,
        out_b1,
        isem0,
        isem1,
        gsem0,
        gsem1,
        osem0,
        osem1,
    ):
        idx_b = (idx_b0, idx_b1)
        rows_b = (rows_b0, rows_b1)
        out_b = (out_b0, out_b1)
        isem = (isem0, isem1)
        gsem = (gsem0, gsem1)
        osem = (osem0, osem1)

        wid = lax.axis_index("s") * nc + lax.axis_index("c")
        base_b = wid * b_per_w
        base_i = base_b * HWP
        iota16 = lax.iota(jnp.int32, 16)

        def start_idx(s, p):
            pltpu.async_copy(
                idx_hbm.at[pl.ds(base_i + s * CHUNK, CHUNK)], idx_b[p], isem[p]
            )

        def start_gather(s, p):
            del s
            pltpu.async_copy(table_hbm.at[idx_b[p]], rows_b[p], gsem[p])

        def wait_idx(p):
            pltpu.make_async_copy(
                idx_hbm.at[pl.ds(0, CHUNK)], idx_b[p], isem[p]
            ).wait()

        def wait_gather(p):
            pltpu.make_async_copy(
                table_hbm.at[idx_b[p]], rows_b[p], gsem[p]
            ).wait()

        def start_out(bb, q):
            pltpu.async_copy(out_b[q], out_hbm.at[bb], osem[q])

        def wait_out(bb, q):
            pltpu.make_async_copy(out_b[q], out_hbm.at[bb], osem[q]).wait()

        def transpose_chunk(rb, ob, c):
            col0 = c * CHUNK

            def jg_body(jg, carry):
                j0 = jg * 16
                rowv = j0 + iota16
                colv = col0 + j0 + iota16
                mask = colv < HW
                for half in range(2):
                    for k in range(16):
                        dvec = (iota16 + k) & 15
                        if half:
                            dvec = dvec | 16
                        v = plsc.load_gather(rb, [rowv, dvec])
                        plsc.store_scatter(ob, [dvec, colv], v, mask=mask)
                return carry

            lax.fori_loop(0, N_JG, jg_body, 0)

        start_idx(0, 0)
        wait_idx(0)
        start_gather(0, 0)
        start_idx(1, 1)

        def loop_body(i, carry):
            for s_local in range(4):
                s = 4 * i + s_local
                p = s_local & 1
                q = (s_local >> 1) & 1
                bb = base_b + 2 * i + (s_local >> 1)

                p1 = (s_local + 1) & 1

                @pl.when(s + 1 < n_chunks)
                def _():
                    wait_idx(p1)
                    start_gather(s + 1, p1)

                @pl.when(s + 2 < n_chunks)
                def _():
                    start_idx(s + 2, p)

                wait_gather(p)
                if s_local in (0, 2):
                    @pl.when(i >= 1)
                    def _():
                        wait_out(bb, q)

                transpose_chunk(rows_b[p], out_b[q], p)
                if s_local in (1, 3):
                    start_out(bb, q)
            return carry

        lax.fori_loop(0, b_per_w // 2, loop_body, 0)

        wait_out(base_b + b_per_w - 2, 0)
        wait_out(base_b + b_per_w - 1, 1)

    return embed_kernel(idx_pad, emb_weight)


def kernel(inputs, emb_weight):
    b, h, w = inputs.shape
    assert h * w == HW
    idx = inputs.reshape(b, HW).astype(jnp.int32)
    idx_pad = jnp.pad(idx, ((0, 0), (0, HWP - HW))).reshape(-1)
    out = _embed_sc(idx_pad, emb_weight, b)
    return out.reshape(b, EMB_DIM, h, w)


# speedup vs baseline: 5.3897x; 1.1657x over previous
import functools

import jax
import jax.numpy as jnp
from jax import lax
from jax.experimental import pallas as pl
from jax.experimental.pallas import tpu as pltpu
from jax.experimental.pallas import tpu_sc as plsc

EMB_DIM = 32
HW = 1300
HWP = 1312
CW = 32
NCHUNK = HWP // CW
BL = 32
ROWS = BL * CW


def _embed_sc(idx_lin, emb_weight, b):
    info = plsc.get_sparse_core_info()
    nc, ns = info.num_cores, info.num_subcores
    nw = nc * ns
    assert b == nw * BL
    tile_r = (HW * EMB_DIM) // 8

    mesh = plsc.VectorSubcoreMesh(core_axis_name="c", subcore_axis_name="s")

    @functools.partial(
        pl.kernel,
        mesh=mesh,
        compiler_params=pltpu.CompilerParams(
            use_tc_tiling_on_sc=False, needs_layout_passes=False
        ),
        out_type=jax.ShapeDtypeStruct((tile_r, 64, 128), jnp.float32),
        scratch_types=[
            pltpu.VMEM((ROWS,), jnp.int32),
            pltpu.VMEM((ROWS,), jnp.int32),
            pltpu.VMEM((ROWS, EMB_DIM), jnp.float32),
            pltpu.VMEM((ROWS, EMB_DIM), jnp.float32),
            pltpu.VMEM((CW * 4, 8, BL), jnp.float32),
            pltpu.SemaphoreType.DMA,
            pltpu.SemaphoreType.DMA,
            pltpu.SemaphoreType.DMA,
            pltpu.SemaphoreType.DMA,
            pltpu.SemaphoreType.DMA,
        ],
    )
    def embed_kernel(
        idx_hbm,
        table_hbm,
        out_hbm,
        idx_b0,
        idx_b1,
        rows_b0,
        rows_b1,
        out_v,
        isem0,
        isem1,
        gsem0,
        gsem1,
        osem,
    ):
        idx_b = (idx_b0, idx_b1)
        rows_b = (rows_b0, rows_b1)
        isem = (isem0, isem1)
        gsem = (gsem0, gsem1)

        wid = lax.axis_index("s") * nc + lax.axis_index("c")
        base_i = wid * (NCHUNK * ROWS)
        tb8 = (wid // 4) * 8
        l0 = (wid % 4) * 32
        iota16 = lax.iota(jnp.int32, 16)

        def start_idx(s, p):
            pltpu.async_copy(
                idx_hbm.at[pl.ds(base_i + s * ROWS, ROWS)], idx_b[p], isem[p]
            )

        def wait_idx(p):
            pltpu.make_async_copy(
                idx_hbm.at[pl.ds(0, ROWS)], idx_b[p], isem[p]
            ).wait()

        def start_gather(p):
            pltpu.async_copy(table_hbm.at[idx_b[p]], rows_b[p], gsem[p])

        def wait_gather(p):
            pltpu.make_async_copy(
                table_hbm.at[idx_b[p]], rows_b[p], gsem[p]
            ).wait()

        def out_dst(s, nr):
            return out_hbm.at[
                pl.ds(s * (CW * 4), nr), pl.ds(tb8, 8), pl.ds(l0, BL)
            ]

        def start_out(s, nr):
            pltpu.async_copy(out_v.at[pl.ds(0, nr), :, :], out_dst(s, nr), osem)

        def wait_out(s, nr):
            pltpu.make_async_copy(
                out_v.at[pl.ds(0, nr), :, :], out_dst(s, nr), osem
            ).wait()

        def transpose_chunk(rb):
            def hw_body(hw_l, carry):
                for dh in range(2):
                    d_ids = dh * 16 + iota16
                    v0 = hw_l * 4 + dh * 2 + (iota16 >> 3)
                    v1 = iota16 & 7
                    for bh in range(2):
                        for k in range(16):
                            b_l = bh * 16 + ((iota16 + k) & 15)
                            r_ids = b_l * CW + hw_l
                            v = plsc.load_gather(rb, [r_ids, d_ids])
                            plsc.store_scatter(out_v, [v0, v1, b_l], v)
                return carry

            lax.fori_loop(0, CW, hw_body, 0)

        start_idx(0, 0)
        wait_idx(0)
        start_gather(0)
        start_idx(1, 1)

        def loop_body(i, carry):
            for sl in range(2):
                s = 2 * i + sl
                p = sl
                wait_idx(1 - p)
                start_gather(1 - p)
                wait_gather(p)

                @pl.when(s + 2 < NCHUNK)
                def _():
                    start_idx(s + 2, p)

                @pl.when(s >= 1)
                def _():
                    wait_out(s - 1, CW * 4)

                transpose_chunk(rows_b[p])
                start_out(s, CW * 4)
            return carry

        lax.fori_loop(0, (NCHUNK - 1) // 2, loop_body, 0)

        wait_gather(0)
        wait_out(NCHUNK - 2, CW * 4)
        transpose_chunk(rows_b[0])
        tail_nr = tile_r - (NCHUNK - 1) * (CW * 4)
        start_out(NCHUNK - 1, tail_nr)
        wait_out(NCHUNK - 1, tail_nr)

    return embed_kernel(idx_lin, emb_weight)


def kernel(inputs, emb_weight):
    b, h, w = inputs.shape
    assert h * w == HW and b == 1024
    idx = inputs.reshape(b, HW).astype(jnp.int32)
    idx_pad = jnp.pad(idx, ((0, 0), (0, HWP - HW)))
    idx_lin = (
        idx_pad.reshape(32, BL, NCHUNK, CW).transpose(0, 2, 1, 3).reshape(-1)
    )
    out3 = _embed_sc(idx_lin, emb_weight, b)
    x = out3.reshape(HW, 4, 8, 8, 128)
    x = x.transpose(2, 4, 1, 3, 0)
    return x.reshape(b, EMB_DIM, h, w)


# speedup vs baseline: 5.6632x; 1.0507x over previous
import functools

import jax
import jax.numpy as jnp
from jax import lax
from jax.experimental import pallas as pl
from jax.experimental.pallas import tpu as pltpu
from jax.experimental.pallas import tpu_sc as plsc

EMB_DIM = 32
HW = 1300
HWP = 1312
NCHUNK = 41
B = 1024
ROWS = B


def _embed_sc(idx_lin, emb_weight):
    info = plsc.get_sparse_core_info()
    nc, ns = info.num_cores, info.num_subcores
    nw = nc * ns
    assert nw * NCHUNK == HWP
    tile_r = (HW * EMB_DIM) // 8

    mesh = plsc.VectorSubcoreMesh(core_axis_name="c", subcore_axis_name="s")

    @functools.partial(
        pl.kernel,
        mesh=mesh,
        compiler_params=pltpu.CompilerParams(
            use_tc_tiling_on_sc=False, needs_layout_passes=False
        ),
        out_type=jax.ShapeDtypeStruct((tile_r, 64, 128), jnp.float32),
        scratch_types=[
            pltpu.VMEM((ROWS,), jnp.int32),
            pltpu.VMEM((ROWS,), jnp.int32),
            pltpu.VMEM((ROWS, EMB_DIM), jnp.float32),
            pltpu.VMEM((ROWS, EMB_DIM), jnp.float32),
            pltpu.VMEM((4, 64, 128), jnp.float32),
            pltpu.SemaphoreType.DMA,
            pltpu.SemaphoreType.DMA,
            pltpu.SemaphoreType.DMA,
            pltpu.SemaphoreType.DMA,
            pltpu.SemaphoreType.DMA,
        ],
    )
    def embed_kernel(
        idx_hbm,
        table_hbm,
        out_hbm,
        idx_b0,
        idx_b1,
        rows_b0,
        rows_b1,
        out_v,
        isem0,
        isem1,
        gsem0,
        gsem1,
        osem,
    ):
        idx_b = (idx_b0, idx_b1)
        rows_b = (rows_b0, rows_b1)
        isem = (isem0, isem1)
        gsem = (gsem0, gsem1)

        wid = lax.axis_index("s") * nc + lax.axis_index("c")
        base_i = wid * (NCHUNK * ROWS)
        hw0 = wid * NCHUNK
        iota16 = lax.iota(jnp.int32, 16)

        def start_idx(s, p):
            pltpu.async_copy(
                idx_hbm.at[pl.ds(base_i + s * ROWS, ROWS)], idx_b[p], isem[p]
            )

        def wait_idx(p):
            pltpu.make_async_copy(
                idx_hbm.at[pl.ds(0, ROWS)], idx_b[p], isem[p]
            ).wait()

        def start_gather(p):
            pltpu.async_copy(table_hbm.at[idx_b[p]], rows_b[p], gsem[p])

        def wait_gather(p):
            pltpu.make_async_copy(
                table_hbm.at[idx_b[p]], rows_b[p], gsem[p]
            ).wait()

        def out_dst(s):
            return out_hbm.at[pl.ds((hw0 + s) * 4, 4), :, :]

        def start_out(s):
            pltpu.async_copy(out_v, out_dst(s), osem)

        def wait_out(s):
            pltpu.make_async_copy(out_v, out_dst(s), osem).wait()

        def transpose_chunk(rb):
            def bh_body(bh, carry):
                r_ids = bh * 16 + iota16
                v2 = (bh & 7) * 16 + iota16
                tb8 = (bh // 8) * 8
                for dh in range(2):
                    for k in range(16):
                        dvec = (iota16 + k) & 15
                        d_ids = dh * 16 + dvec
                        v = plsc.load_gather(rb, [r_ids, d_ids])
                        v0 = dh * 2 + (dvec >> 3)
                        v1 = tb8 + (dvec & 7)
                        plsc.store_scatter(out_v, [v0, v1, v2], v)
                return carry

            lax.fori_loop(0, ROWS // 16, bh_body, 0)

        valid_chunks = jnp.minimum(
            jnp.maximum(HW - hw0, 0), NCHUNK
        )

        start_idx(0, 0)
        wait_idx(0)
        start_gather(0)
        start_idx(1, 1)

        def loop_body(i, carry):
            for sl in range(2):
                s = 2 * i + sl
                p = sl
                wait_idx(1 - p)
                start_gather(1 - p)
                wait_gather(p)

                @pl.when(s + 2 < NCHUNK)
                def _():
                    start_idx(s + 2, p)

                @pl.when(s < valid_chunks)
                def _():
                    @pl.when(jnp.logical_and(s >= 1, s - 1 < valid_chunks))
                    def _():
                        wait_out(s - 1)

                    transpose_chunk(rows_b[p])
                    start_out(s)
            return carry

        lax.fori_loop(0, (NCHUNK - 1) // 2, loop_body, 0)

        s_last = NCHUNK - 1
        wait_gather(0)

        @pl.when(s_last < valid_chunks)
        def _():
            wait_out(s_last - 1)
            transpose_chunk(rows_b[0])
            start_out(s_last)
            wait_out(s_last)

        @pl.when(
            jnp.logical_and(s_last >= valid_chunks, valid_chunks >= 1)
        )
        def _():
            wait_out(valid_chunks - 1)

    return embed_kernel(idx_lin, emb_weight)


def kernel(inputs, emb_weight):
    b, h, w = inputs.shape
    assert h * w == HW and b == B
    idx = inputs.reshape(b, HW).astype(jnp.int32)
    idx_pad = jnp.pad(idx, ((0, 0), (0, HWP - HW)))
    idx_lin = idx_pad.T.reshape(-1)
    out3 = _embed_sc(idx_lin, emb_weight)
    x = out3.reshape(HW, 4, 8, 8, 128)
    x = x.transpose(2, 4, 1, 3, 0)
    return x.reshape(b, EMB_DIM, h, w)
